# baseline (speedup 1.0000x reference)
import functools

import jax
import jax.numpy as jnp
from jax import lax
from jax.experimental import pallas as pl
from jax.experimental.pallas import tpu as pltpu
from jax.experimental.pallas import tpu_sc as plsc

NUM_BUCKETS = 1000000
NC, NS, LANES = 2, 16, 16
NUM_WORKERS = NC * NS


def _sc_hash_gather(cur, prev, table):
    n = cur.shape[0]
    e = table.shape[1]
    b_per_w = n // NUM_WORKERS
    mesh = plsc.VectorSubcoreMesh(core_axis_name="c", subcore_axis_name="s")

    @functools.partial(
        pl.kernel,
        mesh=mesh,
        out_type=jax.ShapeDtypeStruct((n, e), jnp.float32),
        compiler_params=pltpu.CompilerParams(use_tc_tiling_on_sc=False),
        scratch_types=[
            pltpu.VMEM((b_per_w,), jnp.int32),
            pltpu.VMEM((b_per_w,), jnp.int32),
            pltpu.VMEM((b_per_w,), jnp.int32),
            pltpu.VMEM((b_per_w, e), jnp.float32),
            pltpu.SemaphoreType.DMA,
        ],
    )
    def gather_kernel(cur_hbm, prev_hbm, table_hbm, out_hbm,
                      cur_v, prev_v, idx_v, rows_v, sem):
        wid = (lax.axis_index("s") * jnp.int32(NC)
               + lax.axis_index("c")).astype(jnp.int32)
        base = wid * jnp.int32(b_per_w)
        pltpu.sync_copy(cur_hbm.at[pl.ds(base, b_per_w)], cur_v)
        pltpu.sync_copy(prev_hbm.at[pl.ds(base, b_per_w)], prev_v)

        k10 = jnp.full((LANES,), 10, dtype=jnp.int32)
        k100k = jnp.full((LANES,), 100000, dtype=jnp.int32)
        k4729 = jnp.full((LANES,), 4729, dtype=jnp.int32)
        kmod = jnp.full((LANES,), NUM_BUCKETS, dtype=jnp.int32)

        @pl.loop(0, b_per_w, step=LANES)
        def _(i):
            p = prev_v[pl.ds(i, LANES)]
            c = cur_v[pl.ds(i, LANES)]
            h = (p % k10) * k100k + p * k4729 + c
            idx_v[pl.ds(i, LANES)] = h % kmod

        pltpu.async_copy(table_hbm.at[idx_v], rows_v, sem).wait()
        pltpu.sync_copy(rows_v, out_hbm.at[pl.ds(base, b_per_w)])

    return gather_kernel(cur, prev, table)


def _tc_project(rows, w_t, block_rows=2048):
    n, e = rows.shape
    m = w_t.shape[1]

    def mm_body(g_ref, w_ref, o_ref):
        o_ref[...] = jnp.dot(g_ref[...], w_ref[...],
                             preferred_element_type=jnp.float32)

    return pl.pallas_call(
        mm_body,
        out_shape=jax.ShapeDtypeStruct((n, m), jnp.float32),
        grid=(n // block_rows,),
        in_specs=[
            pl.BlockSpec((block_rows, e), lambda i: (i, 0)),
            pl.BlockSpec((e, m), lambda i: (0, 0)),
        ],
        out_specs=pl.BlockSpec((block_rows, m), lambda i: (i, 0)),
    )(rows, w_t)


def kernel(input_ids, embed_weight, proj_weight):
    b, s = input_ids.shape
    m = proj_weight.shape[0]
    ids32 = input_ids.astype(jnp.int32)
    prev32 = jnp.concatenate([ids32[:, :1], ids32[:, :-1]], axis=1)
    cur = ids32.reshape(-1)
    prev = prev32.reshape(-1)
    with jax.enable_x64(False):
        gathered = _sc_hash_gather(cur, prev, embed_weight)
        out = _tc_project(gathered, proj_weight.T)
    return out.reshape(b, s, m)
